# baseline (speedup 1.0000x reference)
import jax
from jax import lax
import jax.numpy as jnp
from jax.experimental import pallas as pl
from jax.experimental.pallas import tpu as pltpu
from jax.experimental.pallas import tpu_sc as plsc

_EMBED = 64
_MODEL = 256
_SCALE = 16.0

_NC = 2
_NS = 16
_NW = _NC * _NS
_CHUNK = 128

_M_TILE = 2048


def _sc_gather(table2, idx2):
    n = idx2.shape[0]
    b_per_w = n // _NW
    n_chunks = b_per_w // _CHUNK
    mesh = plsc.VectorSubcoreMesh(core_axis_name="c", subcore_axis_name="s")

    @pl.kernel(
        out_type=jax.ShapeDtypeStruct((n, 2 * _EMBED), table2.dtype),
        mesh=mesh,
        scratch_types=[
            pltpu.VMEM((_CHUNK,), jnp.int32),
            pltpu.VMEM((_CHUNK, 2 * _EMBED), table2.dtype),
            pltpu.SemaphoreType.DMA,
        ],
    )
    def gather_kernel(table_hbm, idx_hbm, out_hbm, idx_v, rows_v, sem):
        wid = lax.axis_index("s") * _NC + lax.axis_index("c")
        wbase = wid * b_per_w

        @pl.loop(0, n_chunks)
        def _(j):
            base = wbase + j * _CHUNK
            pltpu.sync_copy(idx_hbm.at[pl.ds(base, _CHUNK)], idx_v)
            pltpu.async_copy(table_hbm.at[idx_v], rows_v, sem).wait()
            pltpu.sync_copy(rows_v, out_hbm.at[pl.ds(base, _CHUNK)])

    return gather_kernel(table2, idx2)


def _tc_project(emb, par, w2, b2d):
    n = emb.shape[0]

    def mm_kernel(a_ref, p_ref, w_ref, b_ref, o_ref):
        half = lax.broadcasted_iota(jnp.int32, (_M_TILE, 2 * _EMBED), 1) // _EMBED
        mask = (half == p_ref[...]).astype(jnp.float32)
        acc = jax.lax.dot_general(
            a_ref[...] * mask, w_ref[...], (((1,), (0,)), ((), ())),
            preferred_element_type=jnp.float32,
            precision=jax.lax.Precision.HIGHEST,
        )
        o_ref[...] = (acc + b_ref[...]) * _SCALE

    return pl.pallas_call(
        mm_kernel,
        grid=(n // _M_TILE,),
        in_specs=[
            pl.BlockSpec((_M_TILE, 2 * _EMBED), lambda i: (i, 0)),
            pl.BlockSpec((_M_TILE, 1), lambda i: (i, 0)),
            pl.BlockSpec((2 * _EMBED, _MODEL), lambda i: (0, 0)),
            pl.BlockSpec((1, _MODEL), lambda i: (0, 0)),
        ],
        out_specs=pl.BlockSpec((_M_TILE, _MODEL), lambda i: (i, 0)),
        out_shape=jax.ShapeDtypeStruct((n, _MODEL), jnp.float32),
        compiler_params=pltpu.CompilerParams(dimension_semantics=("parallel",)),
    )(emb, par, w2, b2d)


def kernel(x, table, W, b):
    bsz, seq = x.shape
    idx = x.reshape(bsz * seq).astype(jnp.int32)
    table2 = table.reshape(table.shape[0] // 2, 2 * _EMBED)
    emb = _sc_gather(table2, idx >> 1)
    par = (idx & 1).reshape(-1, 1)
    w2 = jnp.concatenate([W.T, W.T], axis=0)
    out = _tc_project(emb, par, w2, b.reshape(1, _MODEL))
    return out.reshape(bsz, seq, _MODEL)


# speedup vs baseline: 1.2306x; 1.2306x over previous
import jax
from jax import lax
import jax.numpy as jnp
from jax.experimental import pallas as pl
from jax.experimental.pallas import tpu as pltpu
from jax.experimental.pallas import tpu_sc as plsc

_EMBED = 64
_MODEL = 256
_SCALE = 16.0

_NC = 2
_NS = 16
_NW = _NC * _NS
_CHUNK = 256

_M_TILE = 2048


_SUB = 128
_NSUB = _CHUNK // _SUB


def _sc_gather(table2, idx2):
    n = idx2.shape[0]
    b_per_w = n // _NW
    n_chunks = b_per_w // _CHUNK
    mesh = plsc.VectorSubcoreMesh(core_axis_name="c", subcore_axis_name="s")

    @pl.kernel(
        out_type=jax.ShapeDtypeStruct((n, 2 * _EMBED), table2.dtype),
        mesh=mesh,
        scratch_types=[
            pltpu.VMEM((b_per_w,), jnp.int32),
            pltpu.VMEM((_CHUNK, 2 * _EMBED), table2.dtype),
            pltpu.VMEM((_CHUNK, 2 * _EMBED), table2.dtype),
            pltpu.SemaphoreType.DMA,
            pltpu.SemaphoreType.DMA,
        ],
    )
    def gather_kernel(table_hbm, idx_hbm, out_hbm, idx_v, r0, r1, s0, s1):
        wid = lax.axis_index("s") * _NC + lax.axis_index("c")
        wbase = wid * b_per_w

        def start(j, rows, sem):
            for k in range(_NSUB):
                pltpu.async_copy(
                    table_hbm.at[idx_v.at[pl.ds(j * _CHUNK + k * _SUB, _SUB)]],
                    rows.at[pl.ds(k * _SUB, _SUB)],
                    sem,
                )

        def drain(j, rows, sem):
            for k in range(_NSUB):
                pltpu.make_async_copy(
                    table_hbm.at[idx_v.at[pl.ds(j * _CHUNK + k * _SUB, _SUB)]],
                    rows.at[pl.ds(k * _SUB, _SUB)],
                    sem,
                ).wait()

        pltpu.sync_copy(idx_hbm.at[pl.ds(wbase, b_per_w)], idx_v)
        start(0, r0, s0)

        @pl.loop(0, n_chunks // 2)
        def _(jj):
            j = jj * 2
            start(j + 1, r1, s1)
            drain(j, r0, s0)
            pltpu.sync_copy(r0, out_hbm.at[pl.ds(wbase + j * _CHUNK, _CHUNK)])

            @pl.when(j + 2 < n_chunks)
            def _():
                start(j + 2, r0, s0)

            drain(j + 1, r1, s1)
            pltpu.sync_copy(r1, out_hbm.at[pl.ds(wbase + (j + 1) * _CHUNK, _CHUNK)])

    return gather_kernel(table2, idx2)


def _tc_project(emb, par, w2, b2d):
    n = emb.shape[0]

    def mm_kernel(a_ref, p_ref, w_ref, b_ref, o_ref):
        half = lax.broadcasted_iota(jnp.int32, (_M_TILE, 2 * _EMBED), 1) // _EMBED
        mask = (half == p_ref[...]).astype(jnp.float32)
        acc = jax.lax.dot_general(
            a_ref[...] * mask, w_ref[...], (((1,), (0,)), ((), ())),
            preferred_element_type=jnp.float32,
        )
        o_ref[...] = (acc + b_ref[...]) * _SCALE

    return pl.pallas_call(
        mm_kernel,
        grid=(n // _M_TILE,),
        in_specs=[
            pl.BlockSpec((_M_TILE, 2 * _EMBED), lambda i: (i, 0)),
            pl.BlockSpec((_M_TILE, 1), lambda i: (i, 0)),
            pl.BlockSpec((2 * _EMBED, _MODEL), lambda i: (0, 0)),
            pl.BlockSpec((1, _MODEL), lambda i: (0, 0)),
        ],
        out_specs=pl.BlockSpec((_M_TILE, _MODEL), lambda i: (i, 0)),
        out_shape=jax.ShapeDtypeStruct((n, _MODEL), jnp.float32),
        compiler_params=pltpu.CompilerParams(dimension_semantics=("parallel",)),
    )(emb, par, w2, b2d)


def kernel(x, table, W, b):
    bsz, seq = x.shape
    idx = x.reshape(bsz * seq).astype(jnp.int32)
    table2 = table.reshape(table.shape[0] // 2, 2 * _EMBED)
    emb = _sc_gather(table2, idx >> 1)
    par = (idx & 1).reshape(-1, 1)
    w2 = jnp.concatenate([W.T, W.T], axis=0)
    out = _tc_project(emb, par, w2, b.reshape(1, _MODEL))
    return out.reshape(bsz, seq, _MODEL)
